# baseline (speedup 1.0000x reference)
import functools

import jax
import jax.numpy as jnp
from jax import lax
from jax.experimental import pallas as pl
from jax.experimental.pallas import tpu as pltpu
from jax.experimental.pallas import tpu_sc as plsc

B = 16384
D = 32
L = 16
NC = 2
NS = 16
NW = NC * NS
BPW = B // NW
CHUNK = 128
GROUPS = BPW // L
NCH = BPW // CHUNK
NT = D * NCH

V = 1_000_000


def _make_sc_kernel():
    mesh = plsc.VectorSubcoreMesh(core_axis_name="c", subcore_axis_name="s")

    @functools.partial(
        pl.kernel,
        mesh=mesh,
        compiler_params=pltpu.CompilerParams(
            needs_layout_passes=False, use_tc_tiling_on_sc=False),
        out_type=(
            jax.ShapeDtypeStruct((B,), jnp.float32),
            jax.ShapeDtypeStruct((B,), jnp.float32),
        ),
        scratch_types=[
            pltpu.VMEM((BPW,), jnp.int32),
            pltpu.VMEM((BPW,), jnp.int32),
            pltpu.VMEM((BPW,), jnp.int32),
            pltpu.VMEM((NCH, CHUNK), jnp.int32),
            pltpu.VMEM((NCH, CHUNK), jnp.int32),
            pltpu.VMEM((NCH, CHUNK), jnp.int32),
            pltpu.VMEM((BPW * D,), jnp.float32),
            pltpu.VMEM((BPW * D,), jnp.float32),
            pltpu.VMEM((BPW * D,), jnp.float32),
            pltpu.VMEM((BPW,), jnp.float32),
            pltpu.VMEM((BPW,), jnp.float32),
            pltpu.VMEM((BPW,), jnp.float32),
            pltpu.VMEM((BPW,), jnp.float32),
            pltpu.VMEM((BPW,), jnp.float32),
            pltpu.VMEM((L,), jnp.float32),
            pltpu.SemaphoreType.DMA,
        ],
    )
    def k(user_hbm, item_i_hbm, item_j_hbm, eu_hbm, ei_hbm, bu_hbm, bi_hbm,
          gb_hbm, out_i_hbm, out_j_hbm,
          idx_u, idx_i, idx_j, idx_u2, idx_i2, idx_j2, g_u, g_i, g_j,
          bu_v, bi_iv, bi_jv, out_iv, out_jv, gb_v, sem):
        wid = lax.axis_index("s") * NC + lax.axis_index("c")
        base = wid * BPW

        pltpu.sync_copy(user_hbm.at[pl.ds(base, BPW)], idx_u)
        pltpu.sync_copy(item_i_hbm.at[pl.ds(base, BPW)], idx_i)
        pltpu.sync_copy(item_j_hbm.at[pl.ds(base, BPW)], idx_j)
        pltpu.sync_copy(gb_hbm, gb_v)
        for c in range(NCH):
            s = pl.ds(base + c * CHUNK, CHUNK)
            pltpu.sync_copy(user_hbm.at[s], idx_u2.at[c])
            pltpu.sync_copy(item_i_hbm.at[s], idx_i2.at[c])
            pltpu.sync_copy(item_j_hbm.at[s], idx_j2.at[c])

        bias_copies = []
        for c in range(NCH):
            s = pl.ds(c * CHUNK, CHUNK)
            bias_copies.append(pltpu.async_copy(bu_hbm.at[idx_u.at[s]], bu_v.at[s], sem))
            bias_copies.append(pltpu.async_copy(bi_hbm.at[idx_i.at[s]], bi_iv.at[s], sem))
            bias_copies.append(pltpu.async_copy(bi_hbm.at[idx_j.at[s]], bi_jv.at[s], sem))

        def fire(tbl, idx, dst):
            def body(r, carry):
                d = r // NCH
                c = r - d * NCH
                pltpu.async_copy(
                    tbl.at[d, :].at[idx.at[pl.ds(c * CHUNK, CHUNK)]],
                    dst.at[pl.ds(r * CHUNK, CHUNK)],
                    sem,
                )
                return carry
            lax.fori_loop(0, NT, body, 0)

        fire(eu_hbm, idx_u, g_u)
        fire(ei_hbm, idx_i, g_i)
        fire(ei_hbm, idx_j, g_j)

        for cp in bias_copies:
            cp.wait()

        def drain(tbl, idx, dst):
            def body(r, carry):
                d = r // NCH
                c = r - d * NCH
                pltpu.make_async_copy(
                    tbl.at[d, :].at[idx.at[pl.ds(c * CHUNK, CHUNK)]],
                    dst.at[pl.ds(r * CHUNK, CHUNK)],
                    sem,
                ).wait()
                return carry
            lax.fori_loop(0, NT, body, 0)

        drain(eu_hbm, idx_u, g_u)
        drain(ei_hbm, idx_i, g_i)
        drain(ei_hbm, idx_j, g_j)

        gb = gb_v[...]

        def group_body(g, carry):
            bs = pl.ds(g * L, L)
            acc_i = gb + bu_v[bs] + bi_iv[bs]
            acc_j = gb + bu_v[bs] + bi_jv[bs]
            for d in range(D):
                ds_d = pl.ds(d * BPW + g * L, L)
                u = g_u[ds_d]
                acc_i = acc_i + u * g_i[ds_d]
                acc_j = acc_j + u * g_j[ds_d]
            out_iv[bs] = acc_i
            out_jv[bs] = acc_j
            return carry

        lax.fori_loop(0, GROUPS, group_body, 0)

        pltpu.sync_copy(out_iv, out_i_hbm.at[pl.ds(base, BPW)])
        pltpu.sync_copy(out_jv, out_j_hbm.at[pl.ds(base, BPW)])

    return k


_SC_KERNEL = _make_sc_kernel()


def kernel(user, item_i, item_j, embed_user, embed_item, bias_user, bias_item,
           global_bias):
    gb16 = jnp.broadcast_to(global_bias.astype(jnp.float32), (L,))
    return _SC_KERNEL(
        user.astype(jnp.int32),
        item_i.astype(jnp.int32),
        item_j.astype(jnp.int32),
        embed_user.T,
        embed_item.T,
        bias_user.reshape(-1),
        bias_item.reshape(-1),
        gb16,
    )


# speedup vs baseline: 5.6865x; 5.6865x over previous
import functools

import jax
import jax.numpy as jnp
from jax import lax
from jax.experimental import pallas as pl
from jax.experimental.pallas import tpu as pltpu
from jax.experimental.pallas import tpu_sc as plsc

B = 16384
D = 32
L = 16
NC = 2
NS = 16
NW = NC * NS
BPW = B // NW
Q = 128
NQ = BPW // Q
QG = Q // L
V = 1_000_000
VR = (V * D) // 128


def _make_sc_kernel():
    mesh = plsc.VectorSubcoreMesh(core_axis_name="c", subcore_axis_name="s")

    @functools.partial(
        pl.kernel,
        mesh=mesh,
        compiler_params=pltpu.CompilerParams(needs_layout_passes=False),
        out_type=(
            jax.ShapeDtypeStruct((B,), jnp.float32),
            jax.ShapeDtypeStruct((B,), jnp.float32),
        ),
        scratch_types=[
            pltpu.VMEM((BPW,), jnp.int32),
            pltpu.VMEM((BPW,), jnp.int32),
            pltpu.VMEM((BPW,), jnp.int32),
            pltpu.VMEM((BPW,), jnp.int32),
            pltpu.VMEM((BPW,), jnp.int32),
            pltpu.VMEM((BPW,), jnp.int32),
            [[pltpu.VMEM((Q, 128), jnp.float32) for _ in range(3)]
             for _ in range(2)],
            pltpu.VMEM((BPW,), jnp.float32),
            pltpu.VMEM((BPW,), jnp.float32),
            pltpu.VMEM((BPW,), jnp.float32),
            pltpu.VMEM((BPW,), jnp.float32),
            pltpu.VMEM((BPW,), jnp.float32),
            pltpu.VMEM((L,), jnp.float32),
            pltpu.SemaphoreType.DMA,
        ],
    )
    def k(user_hbm, item_i_hbm, item_j_hbm, eu_hbm, ei_hbm, bu_hbm, bi_hbm,
          gb_hbm, out_i_hbm, out_j_hbm,
          idx_u, idx_i, idx_j, row_u, row_i, row_j, bufs,
          bu_v, bi_iv, bi_jv, out_iv, out_jv, gb_v, sem):
        wid = lax.axis_index("s") * NC + lax.axis_index("c")
        base = wid * BPW

        pltpu.sync_copy(user_hbm.at[pl.ds(base, BPW)], idx_u)
        pltpu.sync_copy(item_i_hbm.at[pl.ds(base, BPW)], idx_i)
        pltpu.sync_copy(item_j_hbm.at[pl.ds(base, BPW)], idx_j)
        pltpu.sync_copy(gb_hbm, gb_v)

        bias_copies = []
        for c in range(NQ):
            s = pl.ds(c * Q, Q)
            bias_copies.append(pltpu.async_copy(bu_hbm.at[idx_u.at[s]], bu_v.at[s], sem))
            bias_copies.append(pltpu.async_copy(bi_hbm.at[idx_i.at[s]], bi_iv.at[s], sem))
            bias_copies.append(pltpu.async_copy(bi_hbm.at[idx_j.at[s]], bi_jv.at[s], sem))

        def rowgen(g, carry):
            s = pl.ds(g * L, L)
            row_u[s] = idx_u[s] >> 2
            row_i[s] = idx_i[s] >> 2
            row_j[s] = idx_j[s] >> 2
            return carry
        lax.fori_loop(0, BPW // L, rowgen, 0)

        def fire(q, bset):
            s = pl.ds(q * Q, Q)
            return [
                pltpu.async_copy(eu_hbm.at[row_u.at[s]], bufs[bset][0], sem),
                pltpu.async_copy(ei_hbm.at[row_i.at[s]], bufs[bset][1], sem),
                pltpu.async_copy(ei_hbm.at[row_j.at[s]], bufs[bset][2], sem),
            ]

        gb = gb_v[...]

        def compute(q, bset):
            gu, gi, gj = bufs[bset]

            def group_body(g, carry):
                e = q * Q + g * L
                bs = pl.ds(e, L)
                lb = g * L + lax.iota(jnp.int32, L)
                cu = (idx_u[bs] & 3) << 5
                ci = (idx_i[bs] & 3) << 5
                cj = (idx_j[bs] & 3) << 5
                acc_i = gb + bu_v[bs] + bi_iv[bs]
                acc_j = gb + bu_v[bs] + bi_jv[bs]
                for d in range(D):
                    u = plsc.load_gather(gu, [lb, cu + d])
                    vi = plsc.load_gather(gi, [lb, ci + d])
                    vj = plsc.load_gather(gj, [lb, cj + d])
                    acc_i = acc_i + u * vi
                    acc_j = acc_j + u * vj
                out_iv[bs] = acc_i
                out_jv[bs] = acc_j
                return carry

            lax.fori_loop(0, QG, group_body, 0)

        cps = fire(0, 0)
        nxt = fire(1, 1)
        for q in range(NQ):
            for cp in cps:
                cp.wait()
            compute(q, q % 2)
            pending = fire(q + 2, q % 2) if q + 2 < NQ else None
            cps, nxt = nxt, pending

        for cp in bias_copies:
            cp.wait()

        pltpu.sync_copy(out_iv, out_i_hbm.at[pl.ds(base, BPW)])
        pltpu.sync_copy(out_jv, out_j_hbm.at[pl.ds(base, BPW)])

    return k


_SC_KERNEL = _make_sc_kernel()


def kernel(user, item_i, item_j, embed_user, embed_item, bias_user, bias_item,
           global_bias):
    gb16 = jnp.broadcast_to(global_bias.astype(jnp.float32), (L,))
    return _SC_KERNEL(
        user.astype(jnp.int32),
        item_i.astype(jnp.int32),
        item_j.astype(jnp.int32),
        embed_user.reshape(VR, 128),
        embed_item.reshape(VR, 128),
        bias_user.reshape(-1),
        bias_item.reshape(-1),
        gb16,
    )
